# Initial kernel scaffold; baseline (speedup 1.0000x reference)
import jax
import jax.numpy as jnp
from jax.experimental import pallas as pl


def kernel(x, edge_index, batch, W1, b1, W2, b2, Wfc, bfc):
    raise NotImplementedError("write your pallas kernel here")


# speedup vs baseline: 9.2561x; 9.2561x over previous
import functools

import jax
import jax.numpy as jnp
from jax import lax
from jax.experimental import pallas as pl
from jax.experimental.pallas import tpu as pltpu
from jax.experimental.pallas import tpu_sc as plsc

_N = 10000
_E = 160000
_D = 256
_DH = 128
_DOUT = 128
_G = 64
_NP = 10240
_EP = 163840
_NC = 2
_NS = 16
_CHUNK = 128
_IBLK = 16
_RPT = _NP // _NS
_DEG_CHUNKS = _EP // (_NC * _NS * _CHUNK)
_AGG_CHUNKS = _EP // (_NS * _CHUNK)
_BLK = 1024
_GRID = _NP // _BLK

def _deg_body(dst_hbm, deg_hbm, dstv, ebuf, degs):
    c = lax.axis_index("c")
    s = lax.axis_index("s")
    wid = c * _NS + s
    pltpu.sync_copy(dst_hbm.at[wid], dstv)
    zero = jnp.zeros((16,), jnp.float32)

    def _zero_row(i, _):
        ebuf[i, :] = zero
        return 0

    lax.fori_loop(0, _CHUNK, _zero_row, 0)
    for k in range(_RPT // _CHUNK):
        pltpu.sync_copy(ebuf, degs.at[pl.ds(s * _RPT + k * _CHUNK, _CHUNK)])
    e1 = jnp.where(lax.iota(jnp.int32, 16) == 0, 1.0, 0.0).astype(jnp.float32)

    def _e1_row(i, _):
        ebuf[i, :] = e1
        return 0

    lax.fori_loop(0, _CHUNK, _e1_row, 0)
    plsc.subcore_barrier()

    def _chunk(j, _):
        pltpu.sync_copy(ebuf, degs.at[dstv.at[j]], add=True)
        return 0

    lax.fori_loop(0, _DEG_CHUNKS, _chunk, 0)
    plsc.subcore_barrier()
    pltpu.sync_copy(degs.at[pl.ds(s * _RPT, _RPT)],
                    deg_hbm.at[c, pl.ds(s * _RPT, _RPT)])


def _agg_body(y_hbm, src_hbm, dst_hbm, agg_hbm,
              srcv, dstv, gbuf, aggs, sem0, sem1):
    c = lax.axis_index("c")
    s = lax.axis_index("s")
    zero = jnp.zeros((16,), jnp.float32)

    def _zero_row(i, _):
        r = i // (_DH // 16)
        q = lax.rem(i, _DH // 16)
        gbuf[0, r, pl.ds(q * 16, 16)] = zero
        return 0

    lax.fori_loop(0, _CHUNK * (_DH // 16), _zero_row, 0)
    for k in range(_RPT // _CHUNK):
        pltpu.sync_copy(gbuf.at[0],
                        aggs.at[pl.ds(s * _RPT + k * _CHUNK, _CHUNK)])
    plsc.subcore_barrier()

    def _block(b, _):
        pltpu.sync_copy(src_hbm.at[c, s, pl.ds(b * _IBLK, _IBLK)], srcv)
        pltpu.sync_copy(dst_hbm.at[s, pl.ds(b * _IBLK, _IBLK)], dstv)
        pltpu.async_copy(y_hbm.at[srcv.at[0]], gbuf.at[0], sem0)

        def _pair(p, _):
            j0 = p * 2
            pltpu.async_copy(y_hbm.at[srcv.at[j0 + 1]], gbuf.at[1], sem1)
            pltpu.make_async_copy(y_hbm.at[srcv.at[j0]], gbuf.at[0],
                                  sem0).wait()
            pltpu.sync_copy(gbuf.at[0], aggs.at[dstv.at[j0]], add=True)

            @pl.when(j0 + 2 < _IBLK)
            def _():
                pltpu.async_copy(y_hbm.at[srcv.at[j0 + 2]], gbuf.at[0], sem0)

            pltpu.make_async_copy(y_hbm.at[srcv.at[j0 + 1]], gbuf.at[1],
                                  sem1).wait()
            pltpu.sync_copy(gbuf.at[1], aggs.at[dstv.at[j0 + 1]], add=True)
            return 0

        lax.fori_loop(0, _IBLK // 2, _pair, 0)
        return 0

    lax.fori_loop(0, _AGG_CHUNKS // _IBLK, _block, 0)
    plsc.subcore_barrier()
    pltpu.sync_copy(aggs.at[pl.ds(s * _RPT, _RPT)],
                    agg_hbm.at[c, pl.ds(s * _RPT, _RPT)])


@functools.cache
def _sc_kernels():
    mesh = plsc.VectorSubcoreMesh(core_axis_name="c", subcore_axis_name="s")
    deg = pl.kernel(
        _deg_body,
        out_type=jax.ShapeDtypeStruct((_NC, _NP, 16), jnp.float32),
        mesh=mesh,
        scratch_types=[
            pltpu.VMEM((_DEG_CHUNKS, _CHUNK), jnp.int32),
            pltpu.VMEM((_CHUNK, 16), jnp.float32),
            pltpu.VMEM_SHARED((_NP, 16), jnp.float32),
        ],
    )
    agg = pl.kernel(
        _agg_body,
        out_type=jax.ShapeDtypeStruct((_NC, _NP, _DH), jnp.float32),
        mesh=mesh,
        scratch_types=[
            pltpu.VMEM((_IBLK, _CHUNK), jnp.int32),
            pltpu.VMEM((_IBLK, _CHUNK), jnp.int32),
            pltpu.VMEM((2, _CHUNK, _DH), jnp.float32),
            pltpu.VMEM_SHARED((_NP, _DH), jnp.float32),
            pltpu.SemaphoreType.DMA,
            pltpu.SemaphoreType.DMA,
        ],
    )
    return deg, agg


def _tc1_body(deg_ref, x_ref, w_ref, y_ref, dinv_ref):
    deg = deg_ref[0, :, 0:1] + deg_ref[1, :, 0:1] + 1.0
    dinv = lax.rsqrt(deg)
    xw = jnp.dot(x_ref[...], w_ref[...], preferred_element_type=jnp.float32)
    y = xw * dinv
    y_ref[0] = y[:, :_DH]
    y_ref[1] = y[:, _DH:]
    dinv_ref[...] = dinv


def _tc1(degp, x_p, w1):
    return pl.pallas_call(
        _tc1_body,
        grid=(_GRID,),
        in_specs=[
            pl.BlockSpec((_NC, _BLK, 16), lambda i: (0, i, 0)),
            pl.BlockSpec((_BLK, _D), lambda i: (i, 0)),
            pl.BlockSpec((_D, _D), lambda i: (0, 0)),
        ],
        out_specs=[
            pl.BlockSpec((_NC, _BLK, _DH), lambda i: (0, i, 0)),
            pl.BlockSpec((_BLK, 1), lambda i: (i, 0)),
        ],
        out_shape=[
            jax.ShapeDtypeStruct((_NC, _NP, _DH), jnp.float32),
            jax.ShapeDtypeStruct((_NP, 1), jnp.float32),
        ],
    )(degp, x_p, w1)


def _tc2_body(agg_ref, y_ref, dinv_ref, b_ref, w_ref, out_ref):
    dinv = dinv_ref[...]
    pre = jnp.concatenate([agg_ref[0] + y_ref[0], agg_ref[1] + y_ref[1]],
                          axis=1)
    h = jnp.maximum(pre * dinv + b_ref[...], 0.0)
    z = jnp.dot(h, w_ref[...], preferred_element_type=jnp.float32)
    y2 = z * dinv
    out_ref[0] = y2[:, :_DH]
    out_ref[1] = y2[:, _DH:]


def _tc2(agg1, y1, dinv, b1r, w2):
    return pl.pallas_call(
        _tc2_body,
        grid=(_GRID,),
        in_specs=[
            pl.BlockSpec((_NC, _BLK, _DH), lambda i: (0, i, 0)),
            pl.BlockSpec((_NC, _BLK, _DH), lambda i: (0, i, 0)),
            pl.BlockSpec((_BLK, 1), lambda i: (i, 0)),
            pl.BlockSpec((1, _D), lambda i: (0, 0)),
            pl.BlockSpec((_D, _D), lambda i: (0, 0)),
        ],
        out_specs=pl.BlockSpec((_NC, _BLK, _DH), lambda i: (0, i, 0)),
        out_shape=jax.ShapeDtypeStruct((_NC, _NP, _DH), jnp.float32),
    )(agg1, y1, dinv, b1r, w2)


def _tc3_body(agg_ref, y_ref, dinv_ref, b_ref, batch_ref, wfc_ref, bfc_ref,
              out_ref, acc_ref):
    i = pl.program_id(0)
    dinv = dinv_ref[...]
    pre = jnp.concatenate([agg_ref[0] + y_ref[0], agg_ref[1] + y_ref[1]],
                          axis=1)
    h = jnp.maximum(pre * dinv + b_ref[...], 0.0)
    seg = lax.broadcasted_iota(jnp.int32, (_G, _BLK), 0)
    onehot = (batch_ref[...] == seg).astype(jnp.float32)
    ps = jnp.dot(onehot, h, preferred_element_type=jnp.float32)

    @pl.when(i == 0)
    def _():
        acc_ref[...] = ps

    @pl.when(i > 0)
    def _():
        acc_ref[...] = acc_ref[...] + ps

    @pl.when(i == pl.num_programs(0) - 1)
    def _():
        out_ref[...] = (jnp.dot(acc_ref[...], wfc_ref[...],
                                preferred_element_type=jnp.float32)
                        + bfc_ref[...])


def _tc3(agg2, y2, dinv, b2r, batch_p, wfc, bfcr):
    return pl.pallas_call(
        _tc3_body,
        grid=(_GRID,),
        in_specs=[
            pl.BlockSpec((_NC, _BLK, _DH), lambda i: (0, i, 0)),
            pl.BlockSpec((_NC, _BLK, _DH), lambda i: (0, i, 0)),
            pl.BlockSpec((_BLK, 1), lambda i: (i, 0)),
            pl.BlockSpec((1, _D), lambda i: (0, 0)),
            pl.BlockSpec((1, _BLK), lambda i: (0, i)),
            pl.BlockSpec((_D, _DOUT), lambda i: (0, 0)),
            pl.BlockSpec((1, _DOUT), lambda i: (0, 0)),
        ],
        out_specs=pl.BlockSpec((_G, _DOUT), lambda i: (0, 0)),
        out_shape=jax.ShapeDtypeStruct((_G, _DOUT), jnp.float32),
        scratch_shapes=[pltpu.VMEM((_G, _D), jnp.float32)],
    )(agg2, y2, dinv, b2r, batch_p, wfc, bfcr)


def kernel(x, edge_index, batch, W1, b1, W2, b2, Wfc, bfc):
    x_p = jnp.pad(x, ((0, _NP - _N), (0, 0)))
    src = edge_index[0].astype(jnp.int32)
    dst = edge_index[1].astype(jnp.int32)
    src_p = jnp.concatenate([src, jnp.zeros((_EP - _E,), jnp.int32)])
    dst_p = jnp.concatenate([dst, jnp.full((_EP - _E,), _N, jnp.int32)])
    dst_deg = dst_p.reshape(_NC * _NS, _DEG_CHUNKS, _CHUNK)
    src_agg = jnp.stack([src_p, src_p + _NP]).reshape(
        _NC, _NS, _AGG_CHUNKS, _CHUNK)
    dst_agg = dst_p.reshape(_NS, _AGG_CHUNKS, _CHUNK)
    batch_p = jnp.concatenate(
        [batch.astype(jnp.int32), jnp.full((_NP - _N,), _G, jnp.int32)]
    ).reshape(1, _NP)
    b1r = b1.reshape(1, _D)
    b2r = b2.reshape(1, _D)
    bfcr = bfc.reshape(1, _DOUT)

    deg_sc, agg_sc = _sc_kernels()
    degp = deg_sc(dst_deg)
    y1, dinv = _tc1(degp, x_p, W1)
    agg1 = agg_sc(y1.reshape(_NC * _NP, _DH), src_agg, dst_agg)
    y2 = _tc2(agg1, y1, dinv, b1r, W2)
    agg2 = agg_sc(y2.reshape(_NC * _NP, _DH), src_agg, dst_agg)
    return _tc3(agg2, y2, dinv, b2r, batch_p, Wfc, bfcr)


# speedup vs baseline: 9.4867x; 1.0249x over previous
import functools

import jax
import jax.numpy as jnp
from jax import lax
from jax.experimental import pallas as pl
from jax.experimental.pallas import tpu as pltpu
from jax.experimental.pallas import tpu_sc as plsc

_N = 10000
_E = 160000
_D = 256
_DH = 128
_DOUT = 128
_G = 64
_NP = 10240
_EP = 163840
_NC = 2
_NS = 16
_CHUNK = 128
_IBLK = 40
_RPT = _NP // _NS
_DEG_CHUNKS = _EP // (_NC * _NS * _CHUNK)
_AGG_CHUNKS = _EP // (_NS * _CHUNK)
_BLK = 1024
_GRID = _NP // _BLK


def _deg_body(dst_hbm, deg_hbm, dstv, ebuf, degs):
    c = lax.axis_index("c")
    s = lax.axis_index("s")
    wid = c * _NS + s
    pltpu.sync_copy(dst_hbm.at[wid], dstv)
    zero = jnp.zeros((16,), jnp.float32)

    def _zero_row(i, _):
        ebuf[i, :] = zero
        return 0

    lax.fori_loop(0, _CHUNK, _zero_row, 0)
    for k in range(_RPT // _CHUNK):
        pltpu.sync_copy(ebuf, degs.at[pl.ds(s * _RPT + k * _CHUNK, _CHUNK)])
    e1 = jnp.where(lax.iota(jnp.int32, 16) == 0, 1.0, 0.0).astype(jnp.float32)

    def _e1_row(i, _):
        ebuf[i, :] = e1
        return 0

    lax.fori_loop(0, _CHUNK, _e1_row, 0)
    plsc.subcore_barrier()

    def _chunk(j, _):
        pltpu.sync_copy(ebuf, degs.at[dstv.at[j]], add=True)
        return 0

    lax.fori_loop(0, _DEG_CHUNKS, _chunk, 0)
    plsc.subcore_barrier()
    pltpu.sync_copy(degs.at[pl.ds(s * _RPT, _RPT)],
                    deg_hbm.at[c, pl.ds(s * _RPT, _RPT)])


def _agg_body(y_hbm, src_hbm, dst_hbm, agg_hbm,
              srcv, dstv, gbuf, aggs, sem0, sem1):
    c = lax.axis_index("c")
    s = lax.axis_index("s")
    zero = jnp.zeros((16,), jnp.float32)

    def _zero_row(i, _):
        r = i // (_DH // 16)
        q = lax.rem(i, _DH // 16)
        gbuf[0, r, pl.ds(q * 16, 16)] = zero
        return 0

    lax.fori_loop(0, _CHUNK * (_DH // 16), _zero_row, 0)
    for k in range(_RPT // _CHUNK):
        pltpu.sync_copy(gbuf.at[0],
                        aggs.at[pl.ds(s * _RPT + k * _CHUNK, _CHUNK)])
    plsc.subcore_barrier()

    def _block(b, _):
        pltpu.sync_copy(src_hbm.at[c, s, pl.ds(b * _IBLK, _IBLK)], srcv)
        pltpu.sync_copy(dst_hbm.at[s, pl.ds(b * _IBLK, _IBLK)], dstv)
        pltpu.async_copy(y_hbm.at[srcv.at[0]], gbuf.at[0], sem0)

        def _pair(p, _):
            j0 = p * 2
            pltpu.async_copy(y_hbm.at[srcv.at[j0 + 1]], gbuf.at[1], sem1)
            pltpu.make_async_copy(y_hbm.at[srcv.at[j0]], gbuf.at[0],
                                  sem0).wait()
            pltpu.sync_copy(gbuf.at[0], aggs.at[dstv.at[j0]], add=True)

            @pl.when(j0 + 2 < _IBLK)
            def _():
                pltpu.async_copy(y_hbm.at[srcv.at[j0 + 2]], gbuf.at[0], sem0)

            pltpu.make_async_copy(y_hbm.at[srcv.at[j0 + 1]], gbuf.at[1],
                                  sem1).wait()
            pltpu.sync_copy(gbuf.at[1], aggs.at[dstv.at[j0 + 1]], add=True)
            return 0

        lax.fori_loop(0, _IBLK // 2, _pair, 0)
        return 0

    lax.fori_loop(0, _AGG_CHUNKS // _IBLK, _block, 0)
    plsc.subcore_barrier()
    pltpu.sync_copy(aggs.at[pl.ds(s * _RPT, _RPT)],
                    agg_hbm.at[c, pl.ds(s * _RPT, _RPT)])


@functools.cache
def _sc_kernels():
    mesh = plsc.VectorSubcoreMesh(core_axis_name="c", subcore_axis_name="s")
    deg = pl.kernel(
        _deg_body,
        out_type=jax.ShapeDtypeStruct((_NC, _NP, 16), jnp.float32),
        mesh=mesh,
        scratch_types=[
            pltpu.VMEM((_DEG_CHUNKS, _CHUNK), jnp.int32),
            pltpu.VMEM((_CHUNK, 16), jnp.float32),
            pltpu.VMEM_SHARED((_NP, 16), jnp.float32),
        ],
    )
    agg = pl.kernel(
        _agg_body,
        out_type=jax.ShapeDtypeStruct((_NC, _NP, _DH), jnp.float32),
        mesh=mesh,
        scratch_types=[
            pltpu.VMEM((_IBLK, _CHUNK), jnp.int32),
            pltpu.VMEM((_IBLK, _CHUNK), jnp.int32),
            pltpu.VMEM((2, _CHUNK, _DH), jnp.float32),
            pltpu.VMEM_SHARED((_NP, _DH), jnp.float32),
            pltpu.SemaphoreType.DMA,
            pltpu.SemaphoreType.DMA,
        ],
    )
    return deg, agg


def _tc1_body(deg_ref, x_ref, w_ref, y_ref, dinv_ref):
    deg = deg_ref[0, :, 0:1] + deg_ref[1, :, 0:1] + 1.0
    dinv = lax.rsqrt(deg)
    xw = jnp.dot(x_ref[...], w_ref[...], preferred_element_type=jnp.float32)
    y = xw * dinv
    y_ref[0] = y[:, :_DH]
    y_ref[1] = y[:, _DH:]
    dinv_ref[...] = dinv


def _tc1(degp, x_p, w1):
    return pl.pallas_call(
        _tc1_body,
        grid=(_GRID,),
        in_specs=[
            pl.BlockSpec((_NC, _BLK, 16), lambda i: (0, i, 0)),
            pl.BlockSpec((_BLK, _D), lambda i: (i, 0)),
            pl.BlockSpec((_D, _D), lambda i: (0, 0)),
        ],
        out_specs=[
            pl.BlockSpec((_NC, _BLK, _DH), lambda i: (0, i, 0)),
            pl.BlockSpec((_BLK, 1), lambda i: (i, 0)),
        ],
        out_shape=[
            jax.ShapeDtypeStruct((_NC, _NP, _DH), jnp.float32),
            jax.ShapeDtypeStruct((_NP, 1), jnp.float32),
        ],
    )(degp, x_p, w1)


def _tc2_body(agg_ref, y_ref, dinv_ref, b_ref, w_ref, out_ref):
    dinv = dinv_ref[...]
    pre = jnp.concatenate([agg_ref[0] + y_ref[0], agg_ref[1] + y_ref[1]],
                          axis=1)
    h = jnp.maximum(pre * dinv + b_ref[...], 0.0)
    z = jnp.dot(h, w_ref[...], preferred_element_type=jnp.float32)
    y2 = z * dinv
    out_ref[0] = y2[:, :_DH]
    out_ref[1] = y2[:, _DH:]


def _tc2(agg1, y1, dinv, b1r, w2):
    return pl.pallas_call(
        _tc2_body,
        grid=(_GRID,),
        in_specs=[
            pl.BlockSpec((_NC, _BLK, _DH), lambda i: (0, i, 0)),
            pl.BlockSpec((_NC, _BLK, _DH), lambda i: (0, i, 0)),
            pl.BlockSpec((_BLK, 1), lambda i: (i, 0)),
            pl.BlockSpec((1, _D), lambda i: (0, 0)),
            pl.BlockSpec((_D, _D), lambda i: (0, 0)),
        ],
        out_specs=pl.BlockSpec((_NC, _BLK, _DH), lambda i: (0, i, 0)),
        out_shape=jax.ShapeDtypeStruct((_NC, _NP, _DH), jnp.float32),
    )(agg1, y1, dinv, b1r, w2)


def _tc3_body(agg_ref, y_ref, dinv_ref, b_ref, batch_ref, wfc_ref, bfc_ref,
              out_ref, acc_ref):
    i = pl.program_id(0)
    dinv = dinv_ref[...]
    pre = jnp.concatenate([agg_ref[0] + y_ref[0], agg_ref[1] + y_ref[1]],
                          axis=1)
    h = jnp.maximum(pre * dinv + b_ref[...], 0.0)
    seg = lax.broadcasted_iota(jnp.int32, (_G, _BLK), 0)
    onehot = (batch_ref[...] == seg).astype(jnp.float32)
    ps = jnp.dot(onehot, h, preferred_element_type=jnp.float32)

    @pl.when(i == 0)
    def _():
        acc_ref[...] = ps

    @pl.when(i > 0)
    def _():
        acc_ref[...] = acc_ref[...] + ps

    @pl.when(i == pl.num_programs(0) - 1)
    def _():
        out_ref[...] = (jnp.dot(acc_ref[...], wfc_ref[...],
                                preferred_element_type=jnp.float32)
                        + bfc_ref[...])


def _tc3(agg2, y2, dinv, b2r, batch_p, wfc, bfcr):
    return pl.pallas_call(
        _tc3_body,
        grid=(_GRID,),
        in_specs=[
            pl.BlockSpec((_NC, _BLK, _DH), lambda i: (0, i, 0)),
            pl.BlockSpec((_NC, _BLK, _DH), lambda i: (0, i, 0)),
            pl.BlockSpec((_BLK, 1), lambda i: (i, 0)),
            pl.BlockSpec((1, _D), lambda i: (0, 0)),
            pl.BlockSpec((1, _BLK), lambda i: (0, i)),
            pl.BlockSpec((_D, _DOUT), lambda i: (0, 0)),
            pl.BlockSpec((1, _DOUT), lambda i: (0, 0)),
        ],
        out_specs=pl.BlockSpec((_G, _DOUT), lambda i: (0, 0)),
        out_shape=jax.ShapeDtypeStruct((_G, _DOUT), jnp.float32),
        scratch_shapes=[pltpu.VMEM((_G, _D), jnp.float32)],
    )(agg2, y2, dinv, b2r, batch_p, wfc, bfcr)


def kernel(x, edge_index, batch, W1, b1, W2, b2, Wfc, bfc):
    x_p = jnp.pad(x, ((0, _NP - _N), (0, 0)))
    src = edge_index[0].astype(jnp.int32)
    dst = edge_index[1].astype(jnp.int32)
    src_p = jnp.concatenate([src, jnp.zeros((_EP - _E,), jnp.int32)])
    dst_p = jnp.concatenate([dst, jnp.full((_EP - _E,), _N, jnp.int32)])
    dst_deg = dst_p.reshape(_NC * _NS, _DEG_CHUNKS, _CHUNK)
    src_agg = jnp.stack([src_p, src_p + _NP]).reshape(
        _NC, _NS, _AGG_CHUNKS, _CHUNK)
    dst_agg = dst_p.reshape(_NS, _AGG_CHUNKS, _CHUNK)
    batch_p = jnp.concatenate(
        [batch.astype(jnp.int32), jnp.full((_NP - _N,), _G, jnp.int32)]
    ).reshape(1, _NP)
    b1r = b1.reshape(1, _D)
    b2r = b2.reshape(1, _D)
    bfcr = bfc.reshape(1, _DOUT)

    deg_sc, agg_sc = _sc_kernels()
    degp = deg_sc(dst_deg)
    y1, dinv = _tc1(degp, x_p, W1)
    agg1 = agg_sc(y1.reshape(_NC * _NP, _DH), src_agg, dst_agg)
    y2 = _tc2(agg1, y1, dinv, b1r, W2)
    agg2 = agg_sc(y2.reshape(_NC * _NP, _DH), src_agg, dst_agg)
    return _tc3(agg2, y2, dinv, b2r, batch_p, Wfc, bfcr)
